# baseline (speedup 1.0000x reference)
import functools

import jax
import jax.numpy as jnp
from jax import lax
from jax.experimental import pallas as pl
from jax.experimental.pallas import tpu as pltpu
from jax.experimental.pallas import tpu_sc as plsc

N_NODES = 10000
N_EDGES = 320000
D = 128
NC = 2
NS = 16
NW = NC * NS
EC = 64
CH = 158
EPT = CH * EC
EPAD = NW * EPT
NPAD = 10240
NR = NPAD // 128
RPT = NPAD // NS
NRT = NR // NS

_mesh = plsc.VectorSubcoreMesh(
    core_axis_name="c", subcore_axis_name="s", num_cores=NC, num_subcores=NS)

_f32 = jnp.float32
_bf16 = jnp.bfloat16
_i32 = jnp.int32
_sc_params = pltpu.CompilerParams(needs_layout_passes=False,
                                  use_tc_tiling_on_sc=False)


@functools.partial(
    pl.kernel,
    out_type=(
        jax.ShapeDtypeStruct((NW, NR, 128), _f32),
        jax.ShapeDtypeStruct((NC, NPAD, D), _bf16),
        jax.ShapeDtypeStruct((NR, 128), _f32),
        jax.ShapeDtypeStruct((NR, 128), _f32),
    ),
    mesh=_mesh,
    scratch_types=[
        pltpu.VMEM((CH, EC), jnp.int32),
        pltpu.VMEM((CH, EC), jnp.int32),
        pltpu.VMEM((CH, EC), _f32),
        pltpu.VMEM((CH, EC), jnp.int32),
        pltpu.VMEM((CH, EC), _f32),
        pltpu.VMEM((NR, 128), _f32),
        pltpu.VMEM((NR, 128), _f32),
        pltpu.VMEM((1, NR), jnp.int32),
        pltpu.VMEM((NRT, 128), _f32),
        pltpu.VMEM((NRT, 128), _f32),
        pltpu.VMEM((NRT, 128), _f32),
        pltpu.VMEM((EC, D), _bf16),
        pltpu.VMEM((EC, D), _bf16),
        pltpu.VMEM((EC, D), _bf16),
        pltpu.SemaphoreType.DMA,
        pltpu.SemaphoreType.DMA,
        pltpu.SemaphoreType.DMA,
        pltpu.SemaphoreType.DMA,
        pltpu.SemaphoreType.DMA,
        pltpu.SemaphoreType.DMA,
        pltpu.VMEM_SHARED((NR, 128), _f32),
        pltpu.VMEM_SHARED((NR, 128), _f32),
        pltpu.VMEM_SHARED((NPAD, D), _bf16),
    ],
    compiler_params=_sc_params,
)
def _edge_kernel(src_hbm, dst_hbm, ew_hbm, y_hbm,
                 t_out, agg_out, dinv_out, dinv2_out,
                 src_v, dst_v, ew_v, dst2_v, ew2_v, part, dinv_v, idx_v,
                 degr, dinvr, d2r, rows0, rows1, rows2,
                 gs0, gs1, gs2, ss0, ss1, ss2,
                 deg_sh, dinv_sh, acc_sh):
    c = lax.axis_index("c")
    s = lax.axis_index("s")
    wid = s * NC + c
    sib = s * NC + (1 - c)
    pltpu.sync_copy(src_hbm.at[wid], src_v)
    pltpu.sync_copy(dst_hbm.at[wid], dst_v)
    pltpu.sync_copy(ew_hbm.at[wid], ew_v)
    pltpu.sync_copy(dst_hbm.at[sib], dst2_v)
    pltpu.sync_copy(ew_hbm.at[sib], ew2_v)

    def zero_part(r, _):
        for k in range(8):
            part[r, pl.ds(k * 16, 16)] = jnp.zeros((16,), _f32)
        return 0

    lax.fori_loop(0, NR, zero_part, 0)

    def degbody(dref, wref):
        def body(j, _):
            for k in range(EC // 16):
                sl = pl.ds(k * 16, 16)
                d16 = dref[j, sl]
                w16 = wref[j, sl]
                plsc.addupdate_scatter(
                    part, [d16 >> 7, d16 & 127], w16)
            return 0
        lax.fori_loop(0, CH, body, 0)
    degbody(dst_v, ew_v)
    degbody(dst2_v, ew2_v)

    for g in range(NR // 16):
        idx_v[0, pl.ds(g * 16, 16)] = lax.iota(_i32, 16) + g * 16
    def zero_degr(r, _):
        for k in range(8):
            degr[r, pl.ds(k * 16, 16)] = jnp.zeros((16,), _f32)
        return 0
    lax.fori_loop(0, NRT, zero_degr, 0)
    pltpu.sync_copy(degr, deg_sh.at[pl.ds(s * NRT, NRT)])
    plsc.subcore_barrier()
    pltpu.sync_copy(part, deg_sh.at[idx_v.at[0]], add=True)
    plsc.subcore_barrier()

    pltpu.sync_copy(deg_sh.at[pl.ds(s * NRT, NRT)], degr)
    for r in range(NRT):
        for k in range(8):
            sl = pl.ds(k * 16, 16)
            d = degr[r, sl] + 1.0
            h = 0x5F3759DF - (plsc.bitcast(d, _i32) >> 1)
            x = plsc.bitcast(h, _f32)
            for _ in range(3):
                x = x * (1.5 - 0.5 * d * x * x)
            dinvr[r, sl] = x
            d2r[r, sl] = x * x
    pltpu.sync_copy(dinvr, dinv_sh.at[pl.ds(s * NRT, NRT)])

    @pl.when(c == 0)
    def _():
        pltpu.sync_copy(dinvr, dinv_out.at[pl.ds(s * NRT, NRT)])
        pltpu.sync_copy(d2r, dinv2_out.at[pl.ds(s * NRT, NRT)])
    plsc.subcore_barrier()
    pltpu.sync_copy(dinv_sh, dinv_v)

    lax.fori_loop(0, NR, zero_part, 0)

    def tbody(j, _):
        for k in range(EC // 16):
            sl = pl.ds(k * 16, 16)
            s16 = src_v[j, sl]
            d16 = dst_v[j, sl]
            w16 = ew_v[j, sl]
            dv = plsc.load_gather(dinv_v, [d16 >> 7, d16 & 127])
            plsc.addupdate_scatter(part, [s16 >> 7, s16 & 127], w16 * dv)
        return 0
    lax.fori_loop(0, CH, tbody, 0)
    pltpu.sync_copy(part, t_out.at[wid])

    R = (rows0, rows1, rows2)
    GS = (gs0, gs1, gs2)
    SS = (ss0, ss1, ss2)

    def start_gather(j, b):
        pltpu.async_copy(y_hbm.at[src_v.at[j]], R[b], GS[b])

    def wait_gather(j, b):
        pltpu.make_async_copy(y_hbm.at[src_v.at[j]], R[b], GS[b]).wait()

    def start_scatter(j, b):
        pltpu.async_copy(R[b], acc_sh.at[dst_v.at[j]], SS[b], add=True)

    def wait_scatter(j, b):
        pltpu.make_async_copy(R[b], acc_sh.at[dst_v.at[j]], SS[b]).wait()

    def scale(j, b):
        rb = R[b]

        def sbody(g, _):
            sl = pl.ds(g * 16, 16)
            s16 = src_v[j, sl]
            dv = plsc.load_gather(dinv_v, [s16 >> 7, s16 & 127])
            w16 = ew_v[j, sl] * dv
            for l in range(16):
                wv = jnp.full((16,), w16[l], dtype=_f32)
                wb = plsc.pack(wv, wv, format=plsc.PackFormat.INTERLEAVED)
                e = g * 16 + l
                for k in range(D // 32):
                    rb[e, pl.ds(k * 32, 32)] = rb[e, pl.ds(k * 32, 32)] * wb
            return 0
        lax.fori_loop(0, EC // 16, sbody, 0)

    def zrow(i, _):
        for k in range(D // 32):
            rows0[i, pl.ds(k * 32, 32)] = jnp.zeros((32,), _bf16)
        return 0
    lax.fori_loop(0, EC, zrow, 0)
    for i in range(RPT // EC):
        pltpu.sync_copy(rows0, acc_sh.at[pl.ds(s * RPT + i * EC, EC)])
    plsc.subcore_barrier()

    start_gather(0, 0)
    start_gather(1, 1)
    wait_gather(0, 0)
    scale(0, 0)
    start_gather(2, 2)
    start_scatter(0, 0)
    wait_gather(1, 1)
    scale(1, 1)
    wait_scatter(0, 0)
    start_gather(3, 0)
    start_scatter(1, 1)
    wait_gather(2, 2)
    scale(2, 2)
    wait_scatter(1, 1)
    start_gather(4, 1)
    start_scatter(2, 2)

    def steady(g, _):
        for b3 in range(3):
            j = 3 * g + b3
            b = b3
            wait_gather(j, b)
            scale(j, b)
            wait_scatter(j - 1, (b + 2) % 3)
            start_gather(j + 2, (b + 2) % 3)
            start_scatter(j, b)
        return 0
    lax.fori_loop(1, 52, steady, 0)

    wait_gather(156, 0)
    scale(156, 0)
    wait_scatter(155, 2)
    start_scatter(156, 0)
    wait_gather(157, 1)
    scale(157, 1)
    wait_scatter(156, 0)
    start_scatter(157, 1)
    wait_scatter(157, 1)

    plsc.subcore_barrier()
    pltpu.sync_copy(acc_sh.at[pl.ds(s * RPT, RPT)],
                    agg_out.at[c, pl.ds(s * RPT, RPT)])


def _dense_a_body(x_ref, w1_ref, y_ref, xw1_ref):
    xw1 = jnp.dot(x_ref[...], w1_ref[...],
                  preferred_element_type=_f32,
                  precision=lax.Precision.HIGHEST)
    xw1_ref[...] = xw1
    y_ref[...] = xw1.astype(_bf16)


def _dense_a(x, w1):
    return pl.pallas_call(
        _dense_a_body,
        out_shape=(
            jax.ShapeDtypeStruct((N_NODES, D), _bf16),
            jax.ShapeDtypeStruct((N_NODES, D), _f32),
        ),
    )(x, w1)


def _dense_b_body(aggp_ref, xw1_ref, dinv_ref, dinv2_ref, tp_ref, attr_ref,
                  b1_ref, w2_ref, b2_ref, wm_ref, bm_ref, out_ref):
    dinv = dinv_ref[...][:N_NODES]
    dinv2 = dinv2_ref[...][:N_NODES]
    agg = (aggp_ref[0].astype(_f32) + aggp_ref[1].astype(_f32))[:N_NODES]
    out1 = dinv[:, None] * agg + dinv2[:, None] * xw1_ref[...] + b1_ref[...][None, :]
    h1 = jnp.maximum(out1, 0.0)
    t = jnp.sum(tp_ref[...], axis=0)[:N_NODES]
    s = dinv * t + dinv2
    v128 = jnp.dot(s[None, :], h1, preferred_element_type=_f32,
                   precision=lax.Precision.HIGHEST)
    attr = attr_ref[...]
    va = jnp.dot(s[None, :], attr, preferred_element_type=_f32,
                 precision=lax.Precision.HIGHEST)
    vfull = jnp.concatenate([v128, va], axis=1) * (1.0 / N_NODES)
    mean2 = jnp.dot(vfull, w2_ref[...], preferred_element_type=_f32,
                    precision=lax.Precision.HIGHEST) + b2_ref[...][None, :]
    mean_attr = jnp.sum(attr, axis=0)[None, :] * (1.0 / N_NODES)
    gv = jnp.concatenate([mean2, mean_attr], axis=1)
    out_ref[...] = jnp.dot(gv, wm_ref[...], preferred_element_type=_f32,
                           precision=lax.Precision.HIGHEST) + bm_ref[...][None, :]


def _dense_b(agg_part, xw1, dinv, dinv2, t_part, attributes, b1, w2, b2, wm,
             bm):
    return pl.pallas_call(
        _dense_b_body,
        out_shape=jax.ShapeDtypeStruct((1, D), _f32),
    )(agg_part, xw1, dinv, dinv2, t_part, attributes, b1, w2, b2, wm, bm)


def kernel(x, attributes, edge_obj_to_obj, edge_weight, W1, b1, W2, b2, Wm,
           bm):
    src = edge_obj_to_obj[0].astype(jnp.int32)
    dst = edge_obj_to_obj[1].astype(jnp.int32)
    ew = edge_weight.astype(_f32)
    pad = EPAD - N_EDGES
    srcp = jnp.concatenate([src, jnp.zeros((pad,), jnp.int32)]).reshape(
        NW, CH, EC)
    dstp = jnp.concatenate([dst, jnp.zeros((pad,), jnp.int32)]).reshape(
        NW, CH, EC)
    ewp = jnp.concatenate([ew, jnp.zeros((pad,), _f32)]).reshape(NW, CH, EC)

    y, xw1 = _dense_a(x, W1)
    t_part, agg_part, dinv, dinv2 = _edge_kernel(srcp, dstp, ewp, y)
    return _dense_b(agg_part, xw1, dinv.reshape(NPAD), dinv2.reshape(NPAD),
                    t_part.reshape(NW, NPAD), attributes, b1, W2, b2, Wm, bm)
